# Initial kernel scaffold; baseline (speedup 1.0000x reference)
import jax
import jax.numpy as jnp
from jax.experimental import pallas as pl


def kernel(x, edge_index, batch, atom_emb, conv_params, bn_scale, bn_bias, lin1_W, lin1_b, lin2_W, lin2_b):
    raise NotImplementedError("write your pallas kernel here")


# speedup vs baseline: 1.7238x; 1.7238x over previous
import functools

import jax
import jax.numpy as jnp
from jax import lax
from jax.experimental import pallas as pl
from jax.experimental.pallas import tpu as pltpu
from jax.experimental.pallas import tpu_sc as plsc

N = 10000
NP = 10240
E = 160000
EP = 163840
NG = 64
NSUB = 16
NCORE = 2
ER = EP // 128
EPT = EP // (NSUB * NCORE)
NJ = EPT // 128
RPS = NP // NSUB
R = 1280
NBLK = NP // R
KE = 640

_PREC = lax.Precision.HIGHEST


def _dot(a, b, dims):
    return lax.dot_general(a, b, (dims, ((), ())),
                           preferred_element_type=jnp.float32,
                           precision=_PREC)


def _sc_scatter_body(nchunks, h_ref, src_ref, dst_ref, z_ref, out_ref,
                     srcv, dstv, rows, acc):
    c = lax.axis_index("c")
    s = lax.axis_index("s")
    wid = s * NCORE + c
    pltpu.sync_copy(src_ref.at[pl.ds(wid * NJ, NJ)], srcv)
    pltpu.sync_copy(dst_ref.at[pl.ds(wid * NJ, NJ)], dstv)
    for cc in range(nchunks):
        pltpu.sync_copy(z_ref, acc.at[pl.ds(s * RPS, RPS)])
        plsc.subcore_barrier()

        def step(j, carry):
            pltpu.sync_copy(h_ref.at[cc].at[srcv.at[j]], rows)
            pltpu.sync_copy(rows, acc.at[dstv.at[j]], add=True)
            return carry

        lax.fori_loop(0, NJ, step, 0, unroll=False)
        plsc.subcore_barrier()
        for co in range(NCORE):
            @pl.when(c == co)
            def _(cc=cc, co=co):
                pltpu.sync_copy(acc.at[pl.ds(s * RPS, RPS)],
                                out_ref.at[co].at[cc].at[pl.ds(s * RPS, RPS)])
        plsc.subcore_barrier()


def _sc_scatter(h3, src2, dst2, zrows):
    nchunks = h3.shape[0]
    mesh = plsc.VectorSubcoreMesh(core_axis_name="c", subcore_axis_name="s")
    kfn = pl.kernel(
        functools.partial(_sc_scatter_body, nchunks),
        out_type=jax.ShapeDtypeStruct((NCORE, nchunks, NP, 128), jnp.float32),
        mesh=mesh,
        scratch_types=[
            pltpu.VMEM((NJ, 128), jnp.int32),
            pltpu.VMEM((NJ, 128), jnp.int32),
            pltpu.VMEM((128, 128), jnp.float32),
            pltpu.VMEM_SHARED((NP, 128), jnp.float32),
        ],
    )
    return kfn(h3, src2, dst2, zrows)


def _enc_body(xb_ref, emb_ref, out_ref):
    i = pl.program_id(0)
    xb = xb_ref[...]
    col = lax.broadcasted_iota(jnp.int32, (R, KE), 1)
    oh = jnp.zeros((R, KE), jnp.float32)
    for f in range(9):
        idx = xb[:, f][:, None] + f * 64
        oh = oh + (col == idx).astype(jnp.float32)
    h = _dot(oh, emb_ref[...], ((1,), (0,)))
    row = lax.broadcasted_iota(jnp.int32, (R, 1), 0) + i * R
    h = jnp.where(row < N, h, 0.0)
    for c in range(2):
        out_ref[c] = h[:, c * 128:(c + 1) * 128]


def _encode(xb, emb):
    return pl.pallas_call(
        _enc_body,
        grid=(NBLK,),
        in_specs=[
            pl.BlockSpec((R, 128), lambda i: (i, 0)),
            pl.BlockSpec((KE, 256), lambda i: (0, 0)),
        ],
        out_specs=pl.BlockSpec((2, R, 128), lambda i: (0, i, 0)),
        out_shape=jax.ShapeDtypeStruct((2, NP, 128), jnp.float32),
    )(xb, emb)


def _l1_body(cin, aggp_ref, h_ref, wr_ref, br_ref, wt_ref,
             hp_ref, p1_ref, p2_ref):
    i = pl.program_id(0)
    agg = jnp.concatenate(
        [aggp_ref[0, cc] + aggp_ref[1, cc] for cc in range(cin)], axis=1)
    h = jnp.concatenate([h_ref[cc] for cc in range(cin)], axis=1)
    pre = _dot(agg, wr_ref[...], ((1,), (1,))) \
        + _dot(h, wt_ref[...], ((1,), (1,))) + br_ref[...]
    hp = jnp.maximum(pre, 0.0)
    row = lax.broadcasted_iota(jnp.int32, (R, 1), 0) + i * R
    hp = jnp.where(row < N, hp, 0.0)
    hp_ref[...] = hp
    p1_ref[0] = jnp.sum(hp, axis=0, keepdims=True)
    p2_ref[0] = jnp.sum(hp * hp, axis=0, keepdims=True)


def _layer_mm(aggp, h3, Wr, br2, Wt):
    cin = h3.shape[0]
    dout = Wr.shape[0]
    return pl.pallas_call(
        functools.partial(_l1_body, cin),
        grid=(NBLK,),
        in_specs=[
            pl.BlockSpec((2, cin, R, 128), lambda i: (0, 0, i, 0)),
            pl.BlockSpec((cin, R, 128), lambda i: (0, i, 0)),
            pl.BlockSpec(Wr.shape, lambda i: (0, 0)),
            pl.BlockSpec((1, dout), lambda i: (0, 0)),
            pl.BlockSpec(Wt.shape, lambda i: (0, 0)),
        ],
        out_specs=[
            pl.BlockSpec((R, dout), lambda i: (i, 0)),
            pl.BlockSpec((1, 1, dout), lambda i: (i, 0, 0)),
            pl.BlockSpec((1, 1, dout), lambda i: (i, 0, 0)),
        ],
        out_shape=[
            jax.ShapeDtypeStruct((NP, dout), jnp.float32),
            jax.ShapeDtypeStruct((NBLK, 1, dout), jnp.float32),
            jax.ShapeDtypeStruct((NBLK, 1, dout), jnp.float32),
        ],
    )(aggp, h3, Wr, br2, Wt)


def _bn_body(hp_ref, p1_ref, p2_ref, sc_ref, bb_ref, out_ref):
    i = pl.program_id(0)
    s1 = jnp.sum(p1_ref[...], axis=(0, 1))[None, :]
    s2 = jnp.sum(p2_ref[...], axis=(0, 1))[None, :]
    mu = s1 * (1.0 / N)
    var = s2 * (1.0 / N) - mu * mu
    a = sc_ref[...] * lax.rsqrt(var + 1e-5)
    b = bb_ref[...] - mu * a
    h = hp_ref[...] * a + b
    row = lax.broadcasted_iota(jnp.int32, (R, 1), 0) + i * R
    h = jnp.where(row < N, h, 0.0)
    for cc in range(out_ref.shape[0]):
        out_ref[cc] = h[:, cc * 128:(cc + 1) * 128]


def _bn_apply(hp, p1, p2, bn_scale2, bn_bias2):
    dout = hp.shape[1]
    cout = dout // 128
    return pl.pallas_call(
        _bn_body,
        grid=(NBLK,),
        in_specs=[
            pl.BlockSpec((R, dout), lambda i: (i, 0)),
            pl.BlockSpec((NBLK, 1, dout), lambda i: (0, 0, 0)),
            pl.BlockSpec((NBLK, 1, dout), lambda i: (0, 0, 0)),
            pl.BlockSpec((1, dout), lambda i: (0, 0)),
            pl.BlockSpec((1, dout), lambda i: (0, 0)),
        ],
        out_specs=pl.BlockSpec((cout, R, 128), lambda i: (0, i, 0)),
        out_shape=jax.ShapeDtypeStruct((cout, NP, 128), jnp.float32),
    )(hp, p1, p2, bn_scale2, bn_bias2)


def _pool_body(h_ref, xb_ref, w1_ref, b1_ref, w2_ref, b2_ref, out_ref,
               acc_ref, cnt_ref):
    i = pl.program_id(0)

    @pl.when(i == 0)
    def _():
        acc_ref[...] = jnp.zeros_like(acc_ref)
        cnt_ref[...] = jnp.zeros_like(cnt_ref)

    b = xb_ref[:, 9][:, None]
    g = lax.broadcasted_iota(jnp.int32, (R, NG), 1)
    oh = (g == b).astype(jnp.float32)
    acc_ref[...] += _dot(oh, h_ref[...], ((0,), (0,)))
    cnt_ref[...] += jnp.sum(oh, axis=0, keepdims=True)

    @pl.when(i == NBLK - 1)
    def _():
        cnt = jnp.maximum(cnt_ref[...], 1.0)
        inv = (1.0 / cnt)[0, :][:, None]
        pooled = acc_ref[...] * inv
        o1 = jnp.maximum(_dot(pooled, w1_ref[...], ((1,), (1,)))
                         + b1_ref[...], 0.0)
        o2 = _dot(o1, w2_ref[...], ((1,), (1,))) + b2_ref[0, 0]
        out_ref[...] = o2


def _pool_head(h, xb, w1, b12, w2, b22):
    return pl.pallas_call(
        _pool_body,
        grid=(NBLK,),
        in_specs=[
            pl.BlockSpec((R, 256), lambda i: (i, 0)),
            pl.BlockSpec((R, 128), lambda i: (i, 0)),
            pl.BlockSpec((16, 256), lambda i: (0, 0)),
            pl.BlockSpec((1, 16), lambda i: (0, 0)),
            pl.BlockSpec((128, 16), lambda i: (0, 0)),
            pl.BlockSpec((1, 1), lambda i: (0, 0)),
        ],
        out_specs=pl.BlockSpec((NG, 128), lambda i: (0, 0)),
        out_shape=jax.ShapeDtypeStruct((NG, 128), jnp.float32),
        scratch_shapes=[
            pltpu.VMEM((NG, 256), jnp.float32),
            pltpu.VMEM((1, NG), jnp.float32),
        ],
    )(h, xb, w1, b12, w2, b22)


def kernel(x, edge_index, batch, atom_emb, conv_params, bn_scale, bn_bias,
           lin1_W, lin1_b, lin2_W, lin2_b):
    x = x.astype(jnp.int32)
    batch = batch.astype(jnp.int32)
    src = edge_index[0].astype(jnp.int32)
    dst = edge_index[1].astype(jnp.int32)

    xp = jnp.pad(x, ((0, NP - N), (0, 0)))
    bp = jnp.pad(batch, (0, NP - N), constant_values=NG)[:, None]
    xb = jnp.concatenate(
        [xp, bp, jnp.zeros((NP, 128 - 10), jnp.int32)], axis=1)

    src2 = jnp.pad(src, (0, EP - E), constant_values=N).reshape(ER, 128)
    dst2 = jnp.pad(dst, (0, EP - E), constant_values=N).reshape(ER, 128)
    zrows = jnp.zeros((RPS, 128), jnp.float32)

    emb = jnp.pad(atom_emb.reshape(9 * 64, 256).astype(jnp.float32),
                  ((0, KE - 9 * 64), (0, 0)))
    bn_scale2 = bn_scale[None, :]
    bn_bias2 = bn_bias[None, :]

    h3 = _encode(xb, emb)
    L = len(conv_params)
    hp = None
    for li, (Wr, br, Wt) in enumerate(conv_params):
        aggp = _sc_scatter(h3, src2, dst2, zrows)
        hp, p1, p2 = _layer_mm(aggp, h3, Wr, br[None, :], Wt)
        if li < L - 1:
            h3 = _bn_apply(hp, p1, p2, bn_scale2, bn_bias2)

    w2p = jnp.pad(lin2_W, ((0, 127), (0, 0)))
    out = _pool_head(hp, xb, lin1_W, lin1_b[None, :], w2p, lin2_b[None, :])
    return out[:, :1]
